# baseline (speedup 1.0000x reference)
import jax
import jax.numpy as jnp
from jax.experimental import pallas as pl
from jax.experimental.pallas import tpu as pltpu

B = 16
TOTAL = 32768
IN = 128
H = 128
DIST = 256
ROWS = 1024


def _fused_step(se_ref, traj_ref, nbr_ref, dist_ref, ht_ref,
                w1_ref, w2_ref, w3_ref, be_ref, wih_ref, whh_ref,
                bih_ref, bhh_ref, out_ref):
    base = pl.program_id(0) * ROWS
    rows = jax.lax.broadcasted_iota(jnp.int32, (ROWS, B), 0) + base
    starts = se_ref[0:1, :]
    ends = se_ref[1:2, :]
    onehot = jnp.logical_and(starts <= rows, rows < ends).astype(jnp.float32)

    p = jnp.dot(traj_ref[...], w2_ref[...], preferred_element_type=jnp.float32)
    p = p + be_ref[...]

    emb = jnp.dot(nbr_ref[...], w1_ref[...], preferred_element_type=jnp.float32)
    emb = emb + jnp.dot(dist_ref[...], w3_ref[...], preferred_element_type=jnp.float32)
    emb = emb + jnp.dot(onehot, p, preferred_element_type=jnp.float32)
    x = jnp.maximum(emb, 0.0)

    h = ht_ref[...]
    gi = jnp.dot(x, wih_ref[...], preferred_element_type=jnp.float32) + bih_ref[...]
    gh = jnp.dot(h, whh_ref[...], preferred_element_type=jnp.float32) + bhh_ref[...]
    r = jax.nn.sigmoid(gi[:, 0:H] + gh[:, 0:H])
    z = jax.nn.sigmoid(gi[:, H:2 * H] + gh[:, H:2 * H])
    n = jnp.tanh(gi[:, 2 * H:3 * H] + r * gh[:, 2 * H:3 * H])
    out_ref[...] = (1.0 - z) * n + z * h


def kernel(traj_input, neighbor_t, dist, neighbors_idx_start, neighbors_idx_end,
           ht, W_emb, b_emb, w_ih, w_hh, b_ih, b_hh):
    se = jnp.stack([neighbors_idx_start, neighbors_idx_end]).astype(jnp.int32)
    w1 = W_emb[:, :IN].T
    w2 = W_emb[:, IN:IN + H].T
    w3 = W_emb[:, IN + H:].T
    be = b_emb.reshape(1, H)
    wih = w_ih.T
    whh = w_hh.T
    bih = b_ih.reshape(1, 3 * H)
    bhh = b_hh.reshape(1, 3 * H)

    grid = TOTAL // ROWS
    rep = lambda i: (0, 0)
    out = pl.pallas_call(
        _fused_step,
        grid=(grid,),
        in_specs=[
            pl.BlockSpec((2, B), rep),
            pl.BlockSpec((B, H), rep),
            pl.BlockSpec((ROWS, IN), lambda i: (i, 0)),
            pl.BlockSpec((ROWS, DIST), lambda i: (i, 0)),
            pl.BlockSpec((ROWS, H), lambda i: (i, 0)),
            pl.BlockSpec((IN, H), rep),
            pl.BlockSpec((H, H), rep),
            pl.BlockSpec((DIST, H), rep),
            pl.BlockSpec((1, H), rep),
            pl.BlockSpec((H, 3 * H), rep),
            pl.BlockSpec((H, 3 * H), rep),
            pl.BlockSpec((1, 3 * H), rep),
            pl.BlockSpec((1, 3 * H), rep),
        ],
        out_specs=pl.BlockSpec((ROWS, H), lambda i: (i, 0)),
        out_shape=jax.ShapeDtypeStruct((TOTAL, H), jnp.float32),
        compiler_params=pltpu.CompilerParams(
            dimension_semantics=("parallel",)),
    )(se, traj_input, neighbor_t, dist, ht, w1, w2, w3, be, wih, whh, bih, bhh)
    return out


# speedup vs baseline: 1.2650x; 1.2650x over previous
import jax
import jax.numpy as jnp
from jax.experimental import pallas as pl
from jax.experimental.pallas import tpu as pltpu

B = 16
TOTAL = 32768
IN = 128
H = 128
DIST = 256
ROWS = 4096


def _fused_step(se_ref, traj_ref, nbr_ref, dist_ref, ht_ref,
                w1_ref, w2_ref, w3_ref, be_ref, wih_ref, whh_ref,
                bih_ref, bhh_ref, out_ref):
    base = pl.program_id(0) * ROWS
    rows = jax.lax.broadcasted_iota(jnp.int32, (ROWS, B), 0) + base
    starts = se_ref[0:1, :]
    ends = se_ref[1:2, :]
    onehot = jnp.logical_and(starts <= rows, rows < ends).astype(jnp.float32)

    p = jnp.dot(traj_ref[...], w2_ref[...], preferred_element_type=jnp.float32)
    p = p + be_ref[...]

    bf = jnp.bfloat16
    emb = jnp.dot(nbr_ref[...].astype(bf), w1_ref[...], preferred_element_type=jnp.float32)
    emb = emb + jnp.dot(dist_ref[...].astype(bf), w3_ref[...], preferred_element_type=jnp.float32)
    emb = emb + jnp.dot(onehot, p, preferred_element_type=jnp.float32)
    x = jnp.maximum(emb, 0.0)

    h = ht_ref[...]
    gi = jnp.dot(x.astype(bf), wih_ref[...], preferred_element_type=jnp.float32) + bih_ref[...]
    gh = jnp.dot(h.astype(bf), whh_ref[...], preferred_element_type=jnp.float32) + bhh_ref[...]
    r = jax.nn.sigmoid(gi[:, 0:H] + gh[:, 0:H])
    z = jax.nn.sigmoid(gi[:, H:2 * H] + gh[:, H:2 * H])
    n = jnp.tanh(gi[:, 2 * H:3 * H] + r * gh[:, 2 * H:3 * H])
    out_ref[...] = (1.0 - z) * n + z * h


def kernel(traj_input, neighbor_t, dist, neighbors_idx_start, neighbors_idx_end,
           ht, W_emb, b_emb, w_ih, w_hh, b_ih, b_hh):
    se = jnp.stack([neighbors_idx_start, neighbors_idx_end]).astype(jnp.int32)
    w1 = W_emb[:, :IN].T.astype(jnp.bfloat16)
    w2 = W_emb[:, IN:IN + H].T
    w3 = W_emb[:, IN + H:].T.astype(jnp.bfloat16)
    be = b_emb.reshape(1, H)
    wih = w_ih.T.astype(jnp.bfloat16)
    whh = w_hh.T.astype(jnp.bfloat16)
    bih = b_ih.reshape(1, 3 * H)
    bhh = b_hh.reshape(1, 3 * H)

    grid = TOTAL // ROWS
    rep = lambda i: (0, 0)
    out = pl.pallas_call(
        _fused_step,
        grid=(grid,),
        in_specs=[
            pl.BlockSpec((2, B), rep),
            pl.BlockSpec((B, H), rep),
            pl.BlockSpec((ROWS, IN), lambda i: (i, 0)),
            pl.BlockSpec((ROWS, DIST), lambda i: (i, 0)),
            pl.BlockSpec((ROWS, H), lambda i: (i, 0)),
            pl.BlockSpec((IN, H), rep),
            pl.BlockSpec((H, H), rep),
            pl.BlockSpec((DIST, H), rep),
            pl.BlockSpec((1, H), rep),
            pl.BlockSpec((H, 3 * H), rep),
            pl.BlockSpec((H, 3 * H), rep),
            pl.BlockSpec((1, 3 * H), rep),
            pl.BlockSpec((1, 3 * H), rep),
        ],
        out_specs=pl.BlockSpec((ROWS, H), lambda i: (i, 0)),
        out_shape=jax.ShapeDtypeStruct((TOTAL, H), jnp.float32),
        compiler_params=pltpu.CompilerParams(
            dimension_semantics=("parallel",)),
    )(se, traj_input, neighbor_t, dist, ht, w1, w2, w3, be, wih, whh, bih, bhh)
    return out
